# baseline (speedup 1.0000x reference)
import functools

import jax
import jax.numpy as jnp
from jax import lax
from jax.experimental import pallas as pl
from jax.experimental.pallas import tpu as pltpu
from jax.experimental.pallas import tpu_sc as plsc

BATCH = 16384
HIST = 50
EMBED = 64
TOTAL = BATCH * HIST
NUM_WORKERS = 32
PER_WORKER = TOTAL // NUM_WORKERS
CHUNK = 128
N_CHUNKS = PER_WORKER // CHUNK
NBUF = 8

BBLK = 128
N_BBLK = BATCH // BBLK
BLOCKS_TOTAL = N_BBLK * HIST
BLOCKS_PER_W = BLOCKS_TOTAL // NUM_WORKERS


def _make_gather():
    mesh = plsc.VectorSubcoreMesh(core_axis_name="c", subcore_axis_name="s")

    @functools.partial(
        pl.kernel,
        mesh=mesh,
        compiler_params=pltpu.CompilerParams(use_tc_tiling_on_sc=False),
        out_type=jax.ShapeDtypeStruct((TOTAL, EMBED), jnp.float32),
        scratch_types=[
            pltpu.VMEM((N_CHUNKS, CHUNK), jnp.int32),
            pltpu.VMEM((NBUF, CHUNK, EMBED), jnp.float32),
            pltpu.SemaphoreType.DMA((NBUF,)),
            pltpu.SemaphoreType.DMA((NBUF,)),
        ],
    )
    def gather(table_hbm, idx_hbm, out_hbm, idx_v, rows_v, gsem, osem):
        wid = lax.axis_index("s") * 2 + lax.axis_index("c")
        out_base = wid * PER_WORKER

        pltpu.sync_copy(idx_hbm.at[wid], idx_v)

        def fire_gather(c, b):
            pltpu.async_copy(table_hbm.at[idx_v.at[c]], rows_v.at[b], gsem.at[b])

        for b in range(NBUF):
            fire_gather(b, b)

        def body(g, carry):
            c0 = g * NBUF
            store_descs = []
            for b in range(NBUF):
                c = c0 + b
                pltpu.make_async_copy(
                    table_hbm.at[pl.ds(0, CHUNK)], rows_v.at[b], gsem.at[b]
                ).wait()
                d = pltpu.make_async_copy(
                    rows_v.at[b],
                    out_hbm.at[pl.ds(out_base + c * CHUNK, CHUNK)],
                    osem.at[b],
                )
                d.start()
                store_descs.append(d)
            for b in range(NBUF):
                store_descs[b].wait()
                c_next = c0 + b + NBUF

                @pl.when(c_next < N_CHUNKS)
                def _():
                    fire_gather(c_next, b)

            return carry

        lax.fori_loop(0, N_CHUNKS // NBUF, body, 0)

    return gather


def _make_format():
    mesh = plsc.VectorSubcoreMesh(core_axis_name="c", subcore_axis_name="s")

    @functools.partial(
        pl.kernel,
        mesh=mesh,
        compiler_params=pltpu.CompilerParams(
            use_tc_tiling_on_sc=True, needs_layout_passes=False
        ),
        out_type=jax.ShapeDtypeStruct((HIST, EMBED, BATCH), jnp.float32),
        scratch_types=[
            pltpu.VMEM((2, CHUNK), jnp.int32),
            pltpu.VMEM((2, CHUNK, 128), jnp.float32),
            pltpu.VMEM((2, EMBED, BBLK), jnp.float32),
            pltpu.SemaphoreType.DMA((2,)),
            pltpu.SemaphoreType.DMA((2,)),
        ],
    )
    def fmt(stage_hbm, out_hbm, idx_v, in_v, out_v, gsem, osem):
        wid = lax.axis_index("s") * 2 + lax.axis_index("c")
        lane = lax.iota(jnp.int32, 16)

        def build_idx(blk, buf):
            h = blk % HIST
            b0 = (blk // HIST) * BBLK
            r0 = (b0 * HIST + h) // 2
            for k in range(8):
                idx_v[buf, pl.ds(k * 16, 16)] = r0 + (HIST // 2) * (lane + 16 * k)

        def fire_gather(blk, buf):
            build_idx(blk, buf)
            pltpu.async_copy(
                stage_hbm.at[idx_v.at[buf]], in_v.at[buf], gsem.at[buf]
            )

        def transpose(blk, buf):
            h = blk % HIST
            off = (h % 2) * EMBED

            def frow(f, carry):
                col = jnp.broadcast_to(off + f, (16,)).astype(jnp.int32)
                for k in range(8):
                    v = plsc.load_gather(
                        in_v.at[buf], [lane + 16 * k, col]
                    )
                    out_v[buf, f, pl.ds(k * 16, 16)] = v
                return carry

            lax.fori_loop(0, EMBED, frow, 0)

        def store(blk, buf):
            h = blk % HIST
            b0 = (blk // HIST) * BBLK
            pltpu.async_copy(
                out_v.at[buf],
                out_hbm.at[h, :, pl.ds(b0, BBLK)],
                osem.at[buf],
            )

        blk_base = wid * BLOCKS_PER_W
        fire_gather(blk_base, 0)

        def body(i, carry):
            blk = blk_base + i
            buf = lax.rem(i, 2)
            for p in range(2):
                @pl.when(buf == p)
                def _():
                    @pl.when(i + 1 < BLOCKS_PER_W)
                    def _():
                        fire_gather(blk + 1, 1 - p)
                    pltpu.make_async_copy(
                        stage_hbm.at[pl.ds(0, CHUNK)], in_v.at[p], gsem.at[p]
                    ).wait()
                    @pl.when(i >= 2)
                    def _():
                        pltpu.make_async_copy(
                            out_v.at[p],
                            out_hbm.at[0, :, pl.ds(0, BBLK)],
                            osem.at[p],
                        ).wait()
                    transpose(blk, p)
                    store(blk, p)
            return carry

        lax.fori_loop(0, BLOCKS_PER_W, body, 0)
        for p in range(2):
            pltpu.make_async_copy(
                out_v.at[p], out_hbm.at[0, :, pl.ds(0, BBLK)], osem.at[p]
            ).wait()

    return fmt


_gather_rows = _make_gather()
_format_out = _make_format()


def kernel(gps_idx, weight):
    idx = gps_idx.reshape(NUM_WORKERS, N_CHUNKS, CHUNK).astype(jnp.int32)
    stage = _gather_rows(weight, idx)
    out_t = _format_out(stage.reshape(TOTAL // 2, 2 * EMBED))
    return jnp.transpose(out_t, (2, 0, 1))


# speedup vs baseline: 1.0764x; 1.0764x over previous
import functools

import jax
import jax.numpy as jnp
from jax import lax
from jax.experimental import pallas as pl
from jax.experimental.pallas import tpu as pltpu
from jax.experimental.pallas import tpu_sc as plsc

BATCH = 16384
HIST = 50
EMBED = 64
TOTAL = BATCH * HIST
NUM_WORKERS = 32
PER_WORKER = TOTAL // NUM_WORKERS
CHUNK = 128
N_CHUNKS = PER_WORKER // CHUNK
NBUF = 8

BBLK = 128
N_BBLK = BATCH // BBLK
HPAIR = HIST // 2
PAIRS_TOTAL = N_BBLK * HPAIR
PAIRS_PER_W = PAIRS_TOTAL // NUM_WORKERS


def _make_gather():
    mesh = plsc.VectorSubcoreMesh(core_axis_name="c", subcore_axis_name="s")

    @functools.partial(
        pl.kernel,
        mesh=mesh,
        compiler_params=pltpu.CompilerParams(use_tc_tiling_on_sc=False),
        out_type=jax.ShapeDtypeStruct((TOTAL, EMBED), jnp.float32),
        scratch_types=[
            pltpu.VMEM((N_CHUNKS, CHUNK), jnp.int32),
            pltpu.VMEM((NBUF, CHUNK, EMBED), jnp.float32),
            pltpu.SemaphoreType.DMA((NBUF,)),
            pltpu.SemaphoreType.DMA((NBUF,)),
        ],
    )
    def gather(table_hbm, idx_hbm, out_hbm, idx_v, rows_v, gsem, osem):
        wid = lax.axis_index("s") * 2 + lax.axis_index("c")
        out_base = wid * PER_WORKER

        pltpu.sync_copy(idx_hbm.at[wid], idx_v)

        def fire_gather(c, b):
            pltpu.async_copy(table_hbm.at[idx_v.at[c]], rows_v.at[b], gsem.at[b])

        for b in range(NBUF):
            fire_gather(b, b)

        def body(g, carry):
            c0 = g * NBUF
            store_descs = []
            for b in range(NBUF):
                c = c0 + b
                pltpu.make_async_copy(
                    table_hbm.at[pl.ds(0, CHUNK)], rows_v.at[b], gsem.at[b]
                ).wait()
                d = pltpu.make_async_copy(
                    rows_v.at[b],
                    out_hbm.at[pl.ds(out_base + c * CHUNK, CHUNK)],
                    osem.at[b],
                )
                d.start()
                store_descs.append(d)
            for b in range(NBUF):
                store_descs[b].wait()
                c_next = c0 + b + NBUF

                @pl.when(c_next < N_CHUNKS)
                def _():
                    fire_gather(c_next, b)

            return carry

        lax.fori_loop(0, N_CHUNKS // NBUF, body, 0)

    return gather


def _make_format():
    mesh = plsc.VectorSubcoreMesh(core_axis_name="c", subcore_axis_name="s")

    @functools.partial(
        pl.kernel,
        mesh=mesh,
        compiler_params=pltpu.CompilerParams(
            use_tc_tiling_on_sc=True, needs_layout_passes=False
        ),
        out_type=jax.ShapeDtypeStruct((HIST, EMBED, BATCH), jnp.float32),
        scratch_types=[
            pltpu.VMEM((2, CHUNK), jnp.int32),
            pltpu.VMEM((2, CHUNK, 128), jnp.float32),
            pltpu.VMEM((2, 2, EMBED, BBLK), jnp.float32),
            pltpu.SemaphoreType.DMA((2,)),
            pltpu.SemaphoreType.DMA((2,)),
        ],
    )
    def fmt(stage_hbm, out_hbm, idx_v, in_v, out_v, gsem, osem):
        wid = lax.axis_index("s") * 2 + lax.axis_index("c")
        lane = lax.iota(jnp.int32, 16)

        def fire_gather(q, buf):
            h2 = q % HPAIR
            r0 = (q // HPAIR) * BBLK * HPAIR + h2
            for k in range(8):
                idx_v[buf, pl.ds(k * 16, 16)] = r0 + HPAIR * (lane + 16 * k)
            pltpu.async_copy(
                stage_hbm.at[idx_v.at[buf]], in_v.at[buf], gsem.at[buf]
            )

        def transpose(buf):
            def krow(k, carry):
                row = lane + 16 * k
                kk = k * 16

                def fcol(f, c2):
                    col = jnp.zeros((16,), jnp.int32) + f
                    ve = plsc.load_gather(in_v.at[buf], [row, col])
                    vo = plsc.load_gather(in_v.at[buf], [row, col + EMBED])
                    out_v[buf, 0, f, pl.ds(kk, 16)] = ve
                    out_v[buf, 1, f, pl.ds(kk, 16)] = vo
                    return c2

                lax.fori_loop(0, EMBED, fcol, 0)
                return carry

            lax.fori_loop(0, 8, krow, 0)

        def fire_stores(q, buf):
            h2 = q % HPAIR
            b0 = (q // HPAIR) * BBLK
            for half in range(2):
                pltpu.async_copy(
                    out_v.at[buf, half],
                    out_hbm.at[2 * h2 + half, :, pl.ds(b0, BBLK)],
                    osem.at[buf],
                )

        def drain_gather(buf):
            pltpu.make_async_copy(
                stage_hbm.at[pl.ds(0, CHUNK)], in_v.at[buf], gsem.at[buf]
            ).wait()

        def drain_stores(buf):
            for _ in range(2):
                pltpu.make_async_copy(
                    out_v.at[buf, 0],
                    out_hbm.at[0, :, pl.ds(0, BBLK)],
                    osem.at[buf],
                ).wait()

        q_base = wid * PAIRS_PER_W
        fire_gather(q_base, 0)

        def body(t, carry):
            q0 = q_base + 2 * t

            @pl.when(2 * t + 1 < PAIRS_PER_W)
            def _():
                fire_gather(q0 + 1, 1)

            drain_gather(0)

            @pl.when(t > 0)
            def _():
                drain_stores(0)

            transpose(0)
            fire_stores(q0, 0)

            @pl.when(2 * t + 2 < PAIRS_PER_W)
            def _():
                fire_gather(q0 + 2, 0)

            drain_gather(1)

            @pl.when(t > 0)
            def _():
                drain_stores(1)

            transpose(1)
            fire_stores(q0 + 1, 1)
            return carry

        lax.fori_loop(0, PAIRS_PER_W // 2, body, 0)
        drain_stores(0)
        drain_stores(1)

    return fmt


_gather_rows = _make_gather()
_format_out = _make_format()


def kernel(gps_idx, weight):
    idx = gps_idx.reshape(NUM_WORKERS, N_CHUNKS, CHUNK).astype(jnp.int32)
    stage = _gather_rows(weight, idx)
    out_t = _format_out(stage.reshape(TOTAL // 2, 2 * EMBED))
    return jnp.transpose(out_t, (2, 0, 1))


# speedup vs baseline: 1.1316x; 1.0513x over previous
import functools

import jax
import jax.numpy as jnp
from jax import lax
from jax.experimental import pallas as pl
from jax.experimental.pallas import tpu as pltpu
from jax.experimental.pallas import tpu_sc as plsc

BATCH = 16384
HIST = 50
EMBED = 64
TOTAL = BATCH * HIST
NUM_WORKERS = 32
PER_WORKER = TOTAL // NUM_WORKERS
CHUNK = 128
N_CHUNKS = PER_WORKER // CHUNK
NBUF = 8

BBLK = 128
N_BBLK = BATCH // BBLK
HPAIR = HIST // 2
PAIRS_TOTAL = N_BBLK * HPAIR
PAIRS_PER_W = PAIRS_TOTAL // NUM_WORKERS


def _make_gather():
    mesh = plsc.VectorSubcoreMesh(core_axis_name="c", subcore_axis_name="s")

    @functools.partial(
        pl.kernel,
        mesh=mesh,
        compiler_params=pltpu.CompilerParams(use_tc_tiling_on_sc=False),
        out_type=jax.ShapeDtypeStruct((TOTAL, EMBED), jnp.float32),
        scratch_types=[
            pltpu.VMEM((N_CHUNKS, CHUNK), jnp.int32),
            pltpu.VMEM((NBUF, CHUNK, EMBED), jnp.float32),
            pltpu.SemaphoreType.DMA((NBUF,)),
            pltpu.SemaphoreType.DMA((NBUF,)),
        ],
    )
    def gather(table_hbm, idx_hbm, out_hbm, idx_v, rows_v, gsem, osem):
        wid = lax.axis_index("s") * 2 + lax.axis_index("c")
        out_base = wid * PER_WORKER

        pltpu.sync_copy(idx_hbm.at[wid], idx_v)

        def fire_gather(c, b):
            pltpu.async_copy(table_hbm.at[idx_v.at[c]], rows_v.at[b], gsem.at[b])

        for b in range(NBUF):
            fire_gather(b, b)

        def body(g, carry):
            c0 = g * NBUF
            store_descs = []
            for b in range(NBUF):
                c = c0 + b
                pltpu.make_async_copy(
                    table_hbm.at[pl.ds(0, CHUNK)], rows_v.at[b], gsem.at[b]
                ).wait()
                d = pltpu.make_async_copy(
                    rows_v.at[b],
                    out_hbm.at[pl.ds(out_base + c * CHUNK, CHUNK)],
                    osem.at[b],
                )
                d.start()
                store_descs.append(d)
            for b in range(NBUF):
                store_descs[b].wait()
                c_next = c0 + b + NBUF

                @pl.when(c_next < N_CHUNKS)
                def _():
                    fire_gather(c_next, b)

            return carry

        lax.fori_loop(0, N_CHUNKS // NBUF, body, 0)

    return gather


def _make_format():
    mesh = plsc.VectorSubcoreMesh(core_axis_name="c", subcore_axis_name="s")

    @functools.partial(
        pl.kernel,
        mesh=mesh,
        compiler_params=pltpu.CompilerParams(
            use_tc_tiling_on_sc=True, needs_layout_passes=False
        ),
        out_type=jax.ShapeDtypeStruct((HIST, EMBED, BATCH), jnp.float32),
        scratch_types=[
            pltpu.VMEM((2, CHUNK), jnp.int32),
            pltpu.VMEM((2, CHUNK, 128), jnp.float32),
            pltpu.VMEM((2, 2, EMBED, BBLK + 1), jnp.float32),
            pltpu.SemaphoreType.DMA((2,)),
            pltpu.SemaphoreType.DMA((2,)),
        ],
    )
    def fmt(stage_hbm, out_hbm, idx_v, in_v, out_v, gsem, osem):
        wid = lax.axis_index("s") * 2 + lax.axis_index("c")
        lane = lax.iota(jnp.int32, 16)

        def fire_gather(q, buf):
            h2 = q % HPAIR
            r0 = (q // HPAIR) * BBLK * HPAIR + h2
            for k in range(8):
                idx_v[buf, pl.ds(k * 16, 16)] = r0 + HPAIR * (lane + 16 * k)
            pltpu.async_copy(
                stage_hbm.at[idx_v.at[buf]], in_v.at[buf], gsem.at[buf]
            )

        def transpose(buf):
            def jrow(j, carry):
                cj = jnp.zeros((16,), jnp.int32) + j
                for k in range(8):
                    half = k // 4
                    kk = k % 4
                    v = in_v[buf, j, pl.ds(16 * k, 16)]
                    plsc.store_scatter(
                        out_v.at[buf, half], [16 * kk + lane, cj], v
                    )
                return carry

            lax.fori_loop(0, CHUNK, jrow, 0)

        def fire_stores(q, buf):
            h2 = q % HPAIR
            b0 = (q // HPAIR) * BBLK
            for half in range(2):
                pltpu.async_copy(
                    out_v.at[buf, half, :, pl.ds(0, BBLK)],
                    out_hbm.at[2 * h2 + half, :, pl.ds(b0, BBLK)],
                    osem.at[buf],
                )

        def drain_gather(buf):
            pltpu.make_async_copy(
                stage_hbm.at[pl.ds(0, CHUNK)], in_v.at[buf], gsem.at[buf]
            ).wait()

        def drain_stores(buf):
            for _ in range(2):
                pltpu.make_async_copy(
                    out_v.at[buf, 0, :, pl.ds(0, BBLK)],
                    out_hbm.at[0, :, pl.ds(0, BBLK)],
                    osem.at[buf],
                ).wait()

        q_base = wid * PAIRS_PER_W
        fire_gather(q_base, 0)

        def body(t, carry):
            q0 = q_base + 2 * t

            @pl.when(2 * t + 1 < PAIRS_PER_W)
            def _():
                fire_gather(q0 + 1, 1)

            drain_gather(0)

            @pl.when(t > 0)
            def _():
                drain_stores(0)

            transpose(0)
            fire_stores(q0, 0)

            @pl.when(2 * t + 2 < PAIRS_PER_W)
            def _():
                fire_gather(q0 + 2, 0)

            drain_gather(1)

            @pl.when(t > 0)
            def _():
                drain_stores(1)

            transpose(1)
            fire_stores(q0 + 1, 1)
            return carry

        lax.fori_loop(0, PAIRS_PER_W // 2, body, 0)
        drain_stores(0)
        drain_stores(1)

    return fmt


_gather_rows = _make_gather()
_format_out = _make_format()


def kernel(gps_idx, weight):
    idx = gps_idx.reshape(NUM_WORKERS, N_CHUNKS, CHUNK).astype(jnp.int32)
    stage = _gather_rows(weight, idx)
    out_t = _format_out(stage.reshape(TOTAL // 2, 2 * EMBED))
    return jnp.transpose(out_t, (2, 0, 1))
